# baseline (speedup 1.0000x reference)
import jax
import jax.numpy as jnp
from jax.experimental import pallas as pl


_WEIGHT = 0.0


def _body(h_ref, r_ref, t_ref, o_ref):
    d = h_ref[...] + r_ref[...] - t_ref[...]
    ss = jnp.sum(d * d, axis=-1)
    o_ref[...] = _WEIGHT * jnp.sqrt(ss)


def kernel(h_emb, r_emb, t_emb):
    B, D = h_emb.shape
    blk = 2048
    return pl.pallas_call(
        _body,
        grid=(B // blk,),
        in_specs=[
            pl.BlockSpec((blk, D), lambda i: (i, 0)),
            pl.BlockSpec((blk, D), lambda i: (i, 0)),
            pl.BlockSpec((blk, D), lambda i: (i, 0)),
        ],
        out_specs=pl.BlockSpec((blk,), lambda i: (i,)),
        out_shape=jax.ShapeDtypeStruct((B,), jnp.float32),
    )(h_emb, r_emb, t_emb)
